# Initial kernel scaffold; baseline (speedup 1.0000x reference)
import jax
import jax.numpy as jnp
from jax.experimental import pallas as pl


def kernel(x, Ws1, bs1, Ws2, bs2, W1, b1, W2, b2):
    raise NotImplementedError("write your pallas kernel here")


# speedup vs baseline: 1.3475x; 1.3475x over previous
import functools

import jax
import jax.numpy as jnp
from jax.experimental import pallas as pl
from jax.experimental.pallas import tpu as pltpu

_TOKENS = 4096
_D = 768
_H = 64
_E = 8
_BLK = 256


def _fused_body(x_ref, ws1_ref, bs1_ref, w2a_ref, b2a_ref, w2b_ref, b2b_ref,
                w2c_ref, b2c_ref, g_ref, eps_ref, w1p_ref, b1p_ref, w2p_ref,
                rep_ref, bb2_ref,
                out_ref, ylog_ref, yidx_ref, y_ref, zm_ref, zlv_ref, z_ref):
    xb = x_ref[...]
    h = jnp.maximum(jnp.dot(xb, ws1_ref[...]) + bs1_ref[...], 0.0)
    ylog = jnp.dot(h, w2a_ref[...]) + b2a_ref[...]
    zm = jnp.dot(h, w2b_ref[...]) + b2b_ref[...]
    zlv = jnp.dot(h, w2c_ref[...]) + b2c_ref[...]
    a = ylog + g_ref[...]
    m = jnp.max(a, axis=1, keepdims=True)
    ex = jnp.exp(a - m)
    ysoft = ex / jnp.sum(ex, axis=1, keepdims=True)
    lane = jax.lax.broadcasted_iota(jnp.int32, (_BLK, _E), 1)
    yidx = jnp.min(jnp.where(a == m, lane, _E), axis=1)
    yhard = (lane == yidx[:, None]).astype(jnp.float32)
    y = ysoft + (yhard - ysoft)
    z = zm + jnp.exp(0.5 * zlv) * eps_ref[...]
    gate = yhard * z
    grep = jnp.dot(gate, rep_ref[...])
    xbf = xb.astype(jnp.bfloat16)
    h1 = jnp.dot(xbf, w1p_ref[...], preferred_element_type=jnp.float32)
    h1 = jnp.maximum(h1 + b1p_ref[...], 0.0)
    hg = (h1 * grep).astype(jnp.bfloat16)
    ofc = jnp.dot(hg, w2p_ref[...], preferred_element_type=jnp.float32)
    ofc = ofc + jnp.dot(gate, bb2_ref[...])
    out_ref[...] = xb + ofc
    ylog_ref[...] = ylog
    yidx_ref[...] = yidx[:, None]
    y_ref[...] = y
    zm_ref[...] = zm
    zlv_ref[...] = zlv
    z_ref[...] = z


@functools.partial(jax.jit, static_argnames=())
def kernel(x, Ws1, bs1, Ws2, bs2, W1, b1, W2, b2):
    key = jax.random.key(42)
    k1, k2 = jax.random.split(key)
    u = jax.random.uniform(k1, (_TOKENS, _E), minval=1e-6, maxval=1.0 - 1e-6)
    g = -jnp.log(-jnp.log(u))
    eps = jax.random.normal(k2, (_TOKENS, _E), dtype=jnp.float32)

    w1p = jnp.transpose(W1, (1, 0, 2)).reshape(_D, _E * _H).astype(jnp.bfloat16)
    b1p = b1.reshape(1, _E * _H)
    w2p = W2.reshape(_E * _H, _D).astype(jnp.bfloat16)
    rep = jnp.repeat(jnp.eye(_E, dtype=jnp.float32), _H, axis=1)
    w2a, w2b, w2c = Ws2[:, 0:_E], Ws2[:, _E:2 * _E], Ws2[:, 2 * _E:3 * _E]
    b2a, b2b, b2c = (bs2[0:_E].reshape(1, _E), bs2[_E:2 * _E].reshape(1, _E),
                     bs2[2 * _E:3 * _E].reshape(1, _E))
    bs1r = bs1.reshape(1, _H)

    nblk = _TOKENS // _BLK
    row_spec = lambda w: pl.BlockSpec((_BLK, w), lambda i: (i, 0))
    full = lambda arr: pl.BlockSpec(arr.shape, lambda i: (0,) * arr.ndim)

    outs = pl.pallas_call(
        _fused_body,
        grid=(nblk,),
        in_specs=[
            row_spec(_D),
            full(Ws1), full(bs1r),
            full(w2a), full(b2a), full(w2b), full(b2b), full(w2c), full(b2c),
            row_spec(_E), row_spec(_E),
            full(w1p), full(b1p), full(w2p), full(rep), full(b2),
        ],
        out_specs=[
            row_spec(_D), row_spec(_E), row_spec(1), row_spec(_E),
            row_spec(_E), row_spec(_E), row_spec(_E),
        ],
        out_shape=[
            jax.ShapeDtypeStruct((_TOKENS, _D), jnp.float32),
            jax.ShapeDtypeStruct((_TOKENS, _E), jnp.float32),
            jax.ShapeDtypeStruct((_TOKENS, 1), jnp.int32),
            jax.ShapeDtypeStruct((_TOKENS, _E), jnp.float32),
            jax.ShapeDtypeStruct((_TOKENS, _E), jnp.float32),
            jax.ShapeDtypeStruct((_TOKENS, _E), jnp.float32),
            jax.ShapeDtypeStruct((_TOKENS, _E), jnp.float32),
        ],
    )(x, Ws1, bs1r, w2a, b2a, w2b, b2b, w2c, b2c, g, eps, w1p, b1p, w2p, rep, b2)

    out, ylog, yidx, y, zm, zlv, z = outs
    return (out, ylog, yidx.reshape(_TOKENS), y, zm, zlv, z)


# speedup vs baseline: 2.4771x; 1.8383x over previous
import functools

import jax
import jax.numpy as jnp
from jax.experimental import pallas as pl
from jax.experimental.pallas import tpu as pltpu

_TOKENS = 4096
_D = 768
_H = 64
_E = 8
_BLK = 512


def _fused_body(x_ref, ws1_ref, bs1_ref, w2a_ref, b2a_ref, w2b_ref, b2b_ref,
                w2c_ref, b2c_ref, g_ref, eps_ref, w1p_ref, b1p_ref, w2p_ref,
                rep_ref, bb2_ref,
                out_ref, ylog_ref, yidx_ref, y_ref, zm_ref, zlv_ref, z_ref):
    xb = x_ref[...]
    h = jnp.maximum(jnp.dot(xb, ws1_ref[...]) + bs1_ref[...], 0.0)
    ylog = jnp.dot(h, w2a_ref[...]) + b2a_ref[...]
    zm = jnp.dot(h, w2b_ref[...]) + b2b_ref[...]
    zlv = jnp.dot(h, w2c_ref[...]) + b2c_ref[...]
    a = ylog + g_ref[...]
    m = jnp.max(a, axis=1, keepdims=True)
    ex = jnp.exp(a - m)
    ysoft = ex / jnp.sum(ex, axis=1, keepdims=True)
    lane = jax.lax.broadcasted_iota(jnp.int32, (_BLK, _E), 1)
    yidx = jnp.min(jnp.where(a == m, lane, _E), axis=1)
    yhard = (lane == yidx[:, None]).astype(jnp.float32)
    y = ysoft + (yhard - ysoft)
    z = zm + jnp.exp(0.5 * zlv) * eps_ref[...]
    gate = yhard * z
    grep = jnp.dot(gate, rep_ref[...])
    xbf = xb.astype(jnp.bfloat16)
    h1 = jnp.dot(xbf, w1p_ref[...], preferred_element_type=jnp.float32)
    h1 = jnp.maximum(h1 + b1p_ref[...], 0.0)
    hg = (h1 * grep).astype(jnp.bfloat16)
    ofc = jnp.dot(hg, w2p_ref[...], preferred_element_type=jnp.float32)
    ofc = ofc + jnp.dot(gate, bb2_ref[...])
    out_ref[...] = xb + ofc
    ylog_ref[...] = ylog
    yidx_ref[...] = yidx[:, None]
    y_ref[...] = y
    zm_ref[...] = zm
    zlv_ref[...] = zlv
    z_ref[...] = z


@functools.partial(jax.jit, static_argnames=())
def kernel(x, Ws1, bs1, Ws2, bs2, W1, b1, W2, b2):
    with jax.ensure_compile_time_eval():
        key = jax.random.key(42)
        k1, k2 = jax.random.split(key)
        u = jax.random.uniform(k1, (_TOKENS, _E), minval=1e-6, maxval=1.0 - 1e-6)
        g = -jnp.log(-jnp.log(u))
        eps = jax.random.normal(k2, (_TOKENS, _E), dtype=jnp.float32)

    w1p = jnp.transpose(W1, (1, 0, 2)).reshape(_D, _E * _H).astype(jnp.bfloat16)
    b1p = b1.reshape(1, _E * _H)
    w2p = W2.reshape(_E * _H, _D).astype(jnp.bfloat16)
    rep = jnp.repeat(jnp.eye(_E, dtype=jnp.float32), _H, axis=1)
    w2a, w2b, w2c = Ws2[:, 0:_E], Ws2[:, _E:2 * _E], Ws2[:, 2 * _E:3 * _E]
    b2a, b2b, b2c = (bs2[0:_E].reshape(1, _E), bs2[_E:2 * _E].reshape(1, _E),
                     bs2[2 * _E:3 * _E].reshape(1, _E))
    bs1r = bs1.reshape(1, _H)

    nblk = _TOKENS // _BLK
    row_spec = lambda w: pl.BlockSpec((_BLK, w), lambda i: (i, 0))
    full = lambda arr: pl.BlockSpec(arr.shape, lambda i: (0,) * arr.ndim)

    outs = pl.pallas_call(
        _fused_body,
        grid=(nblk,),
        in_specs=[
            row_spec(_D),
            full(Ws1), full(bs1r),
            full(w2a), full(b2a), full(w2b), full(b2b), full(w2c), full(b2c),
            row_spec(_E), row_spec(_E),
            full(w1p), full(b1p), full(w2p), full(rep), full(b2),
        ],
        out_specs=[
            row_spec(_D), row_spec(_E), row_spec(1), row_spec(_E),
            row_spec(_E), row_spec(_E), row_spec(_E),
        ],
        out_shape=[
            jax.ShapeDtypeStruct((_TOKENS, _D), jnp.float32),
            jax.ShapeDtypeStruct((_TOKENS, _E), jnp.float32),
            jax.ShapeDtypeStruct((_TOKENS, 1), jnp.int32),
            jax.ShapeDtypeStruct((_TOKENS, _E), jnp.float32),
            jax.ShapeDtypeStruct((_TOKENS, _E), jnp.float32),
            jax.ShapeDtypeStruct((_TOKENS, _E), jnp.float32),
            jax.ShapeDtypeStruct((_TOKENS, _E), jnp.float32),
        ],
    )(x, Ws1, bs1r, w2a, b2a, w2b, b2b, w2c, b2c, g, eps, w1p, b1p, w2p, rep, b2)

    out, ylog, yidx, y, zm, zlv, z = outs
    return (out, ylog, yidx.reshape(_TOKENS), y, zm, zlv, z)
